# Initial kernel scaffold; baseline (speedup 1.0000x reference)
import jax
import jax.numpy as jnp
from jax.experimental import pallas as pl


def kernel(x, edge_index, W1, b1, gamma1, beta1, W2, b2, gamma2, beta2, Wc, bc):
    raise NotImplementedError("write your pallas kernel here")


# speedup vs baseline: 18.0142x; 18.0142x over previous
import functools

import jax
import jax.numpy as jnp
from jax import lax
from jax.experimental import pallas as pl
from jax.experimental.pallas import tpu as pltpu
from jax.experimental.pallas import tpu_sc as plsc

N_NODES = 10000
N_EDGES = 320000
D_IN = 128
H1 = 64
H2 = 32
C_OUT = 2

NC = 2
NS = 16
NW = NC * NS
CHUNK = 128
NCHUNK = 80
E_PAD = NW * NCHUNK * CHUNK
NPAD = 10112
ROWS_PER_TILE = NPAD // NS


def _copy_spmem_slice_to_hbm(acc, out_c, vbuf, base):
    done = 0
    while done < ROWS_PER_TILE:
        n = min(CHUNK, ROWS_PER_TILE - done)
        off = base + done
        pltpu.sync_copy(acc.at[pl.ds(off, n)], vbuf.at[pl.ds(0, n)])
        pltpu.sync_copy(vbuf.at[pl.ds(0, n)], out_c.at[pl.ds(off, n)])
        done += n


def _zero_spmem_slice(zeros_hbm, acc, vbuf, base):
    pltpu.sync_copy(zeros_hbm, vbuf)
    done = 0
    while done < ROWS_PER_TILE:
        n = min(CHUNK, ROWS_PER_TILE - done)
        pltpu.sync_copy(vbuf.at[pl.ds(0, n)], acc.at[pl.ds(base + done, n)])
        done += n


def _make_agg(H):
    mesh = plsc.VectorSubcoreMesh(core_axis_name="c", subcore_axis_name="s")

    @functools.partial(
        pl.kernel,
        out_type=jax.ShapeDtypeStruct((NC, NPAD, H), jnp.float32),
        mesh=mesh,
        scratch_types=[
            pltpu.VMEM((NCHUNK, CHUNK), jnp.int32),
            pltpu.VMEM((NCHUNK, CHUNK), jnp.int32),
            pltpu.VMEM((CHUNK, H), jnp.float32),
            pltpu.VMEM((CHUNK, H), jnp.float32),
            pltpu.VMEM_SHARED((NPAD, H), jnp.float32),
            pltpu.SemaphoreType.DMA,
            pltpu.SemaphoreType.DMA,
        ],
        compiler_params=pltpu.CompilerParams(use_tc_tiling_on_sc=False),
    )
    def agg(table, src_i, dst_i, zeros_hbm, out, sidx, didx, bufa, bufb,
            acc, sema, semb):
        c = lax.axis_index("c")
        s = lax.axis_index("s")
        wid = c * NS + s
        base = s * ROWS_PER_TILE
        _zero_spmem_slice(zeros_hbm, acc, bufa, base)
        pltpu.sync_copy(src_i.at[wid], sidx)
        pltpu.sync_copy(dst_i.at[wid], didx)
        plsc.subcore_barrier()

        pltpu.async_copy(table.at[sidx.at[0]], bufa, sema)

        def body(i, carry):
            j = 2 * i
            pltpu.async_copy(table.at[sidx.at[j + 1]], bufb, semb)
            pltpu.make_async_copy(table.at[sidx.at[j]], bufa, sema).wait()
            pltpu.sync_copy(bufa, acc.at[didx.at[j]], add=True)

            @pl.when(j + 2 < NCHUNK)
            def _():
                pltpu.async_copy(table.at[sidx.at[j + 2]], bufa, sema)

            pltpu.make_async_copy(table.at[sidx.at[j + 1]], bufb, semb).wait()
            pltpu.sync_copy(bufb, acc.at[didx.at[j + 1]], add=True)
            return carry

        lax.fori_loop(0, NCHUNK // 2, body, 0)
        plsc.subcore_barrier()
        _copy_spmem_slice_to_hbm(acc, out.at[c], bufa, base)

    return agg


DEGW = 8


def _make_deg():
    mesh = plsc.VectorSubcoreMesh(core_axis_name="c", subcore_axis_name="s")

    @functools.partial(
        pl.kernel,
        out_type=jax.ShapeDtypeStruct((NC, NPAD, DEGW), jnp.float32),
        mesh=mesh,
        scratch_types=[
            pltpu.VMEM((NCHUNK, CHUNK), jnp.int32),
            pltpu.VMEM((CHUNK, DEGW), jnp.float32),
            pltpu.VMEM((CHUNK, DEGW), jnp.float32),
            pltpu.VMEM_SHARED((NPAD, DEGW), jnp.float32),
        ],
        compiler_params=pltpu.CompilerParams(use_tc_tiling_on_sc=False),
    )
    def deg(dst_i, ones_hbm, zeros_hbm, out, didx, ones_v, vbuf, acc):
        c = lax.axis_index("c")
        s = lax.axis_index("s")
        wid = c * NS + s
        base = s * ROWS_PER_TILE
        _zero_spmem_slice(zeros_hbm, acc, vbuf, base)
        pltpu.sync_copy(ones_hbm, ones_v)
        pltpu.sync_copy(dst_i.at[wid], didx)
        plsc.subcore_barrier()

        def body(j, carry):
            pltpu.sync_copy(ones_v, acc.at[didx.at[j]], add=True)
            return carry

        lax.fori_loop(0, NCHUNK, body, 0)
        plsc.subcore_barrier()
        _copy_spmem_slice_to_hbm(acc, out.at[c], vbuf, base)

    return deg


_ROW2 = lambda shape: lax.broadcasted_iota(jnp.int32, shape, 0)


def _tc1_body(x_ref, w1_ref, degp_ref, hn1_ref, dinv_ref):
    deg = degp_ref[0, :, 0:1] + degp_ref[1, :, 0:1] + 1.0
    valid = _ROW2((NPAD, 1)) < N_NODES
    dinv = jnp.where(valid, lax.rsqrt(jnp.maximum(deg, 1e-12)), 0.0)
    h = jnp.dot(x_ref[...], w1_ref[...], preferred_element_type=jnp.float32)
    hn1_ref[...] = h * dinv
    dinv_ref[...] = dinv


def _bn_relu(conv, gamma, beta):
    valid = _ROW2(conv.shape) < N_NODES
    convm = jnp.where(valid, conv, 0.0)
    mean = jnp.sum(convm, axis=0, keepdims=True) / N_NODES
    dev = jnp.where(valid, conv - mean, 0.0)
    var = jnp.sum(dev * dev, axis=0, keepdims=True) / N_NODES
    y = (conv - mean) * lax.rsqrt(var + 1e-5) * gamma + beta
    return jnp.where(valid, jnp.maximum(y, 0.0), 0.0)


def _tc2_body(agg_ref, hn1_ref, dinv_ref, b1_ref, g1_ref, be1_ref, w2_ref,
              hn2_ref):
    dinv = dinv_ref[...]
    conv = (agg_ref[0] + agg_ref[1] + hn1_ref[...]) * dinv + b1_ref[...]
    y = _bn_relu(conv, g1_ref[...], be1_ref[...])
    h2 = jnp.dot(y, w2_ref[...], preferred_element_type=jnp.float32)
    hn2_ref[...] = h2 * dinv


def _tc3_body(agg_ref, hn2_ref, dinv_ref, b2_ref, g2_ref, be2_ref, wc_ref,
              bc_ref, out_ref):
    dinv = dinv_ref[...]
    conv = (agg_ref[0] + agg_ref[1] + hn2_ref[...]) * dinv + b2_ref[...]
    y = _bn_relu(conv, g2_ref[...], be2_ref[...])
    logits = jnp.dot(y, wc_ref[...], preferred_element_type=jnp.float32)
    logits = logits + bc_ref[...]
    m = jnp.max(logits, axis=1, keepdims=True)
    lse = jnp.log(jnp.sum(jnp.exp(logits - m), axis=1, keepdims=True)) + m
    out_ref[...] = logits - lse


_deg_call = _make_deg()
_agg64_call = _make_agg(H1)
_agg32_call = _make_agg(H2)

_tc1_call = pl.pallas_call(
    _tc1_body,
    out_shape=(
        jax.ShapeDtypeStruct((NPAD, H1), jnp.float32),
        jax.ShapeDtypeStruct((NPAD, 1), jnp.float32),
    ),
)

_tc2_call = pl.pallas_call(
    _tc2_body,
    out_shape=jax.ShapeDtypeStruct((NPAD, H2), jnp.float32),
)

_tc3_call = pl.pallas_call(
    _tc3_body,
    out_shape=jax.ShapeDtypeStruct((NPAD, C_OUT), jnp.float32),
)


def kernel(x, edge_index, W1, b1, gamma1, beta1, W2, b2, gamma2, beta2, Wc,
           bc):
    pad = E_PAD - N_EDGES
    fill = jnp.full((pad,), N_NODES, dtype=jnp.int32)
    src = jnp.concatenate([edge_index[0], fill]).reshape(NW, NCHUNK, CHUNK)
    dst = jnp.concatenate([edge_index[1], fill]).reshape(NW, NCHUNK, CHUNK)

    ones1 = jnp.ones((CHUNK, DEGW), jnp.float32)
    zeros1 = jnp.zeros((CHUNK, DEGW), jnp.float32)
    zeros64 = jnp.zeros((CHUNK, H1), jnp.float32)
    zeros32 = jnp.zeros((CHUNK, H2), jnp.float32)

    degp = _deg_call(dst, ones1, zeros1)

    xpad = jnp.pad(x, ((0, NPAD - N_NODES), (0, 0)))
    hn1, dinv = _tc1_call(xpad, W1, degp)

    agg1 = _agg64_call(hn1, src, dst, zeros64)
    hn2 = _tc2_call(agg1, hn1, dinv, b1.reshape(1, H1), gamma1.reshape(1, H1),
                    beta1.reshape(1, H1), W2)

    agg2 = _agg32_call(hn2, src, dst, zeros32)
    out = _tc3_call(agg2, hn2, dinv, b2.reshape(1, H2), gamma2.reshape(1, H2),
                    beta2.reshape(1, H2), Wc, bc.reshape(1, C_OUT))
    return out[:N_NODES]


# speedup vs baseline: 23.0347x; 1.2787x over previous
import functools

import jax
import jax.numpy as jnp
from jax import lax
from jax.experimental import pallas as pl
from jax.experimental.pallas import tpu as pltpu
from jax.experimental.pallas import tpu_sc as plsc

N_NODES = 10000
N_EDGES = 320000
D_IN = 128
H1 = 64
H2 = 32
C_OUT = 2

NC = 2
NS = 16
NW = NC * NS
CHUNK = 128
NCHUNK = 80
E_PAD = NW * NCHUNK * CHUNK
TOTAL_CHUNKS = E_PAD // CHUNK
NBUF = 4
NC0 = 80
NC1 = 80
NCMAX = max(NC0, NC1)
SLACK = NCMAX
NPAD = 10112
ROWS_PER_TILE = NPAD // NS


def _copy_spmem_slice_to_hbm(acc, out_c, vbuf, base):
    done = 0
    while done < ROWS_PER_TILE:
        n = min(CHUNK, ROWS_PER_TILE - done)
        off = base + done
        pltpu.sync_copy(acc.at[pl.ds(off, n)], vbuf.at[pl.ds(0, n)])
        pltpu.sync_copy(vbuf.at[pl.ds(0, n)], out_c.at[pl.ds(off, n)])
        done += n


def _zero_spmem_slice(zeros_hbm, acc, vbuf, base):
    pltpu.sync_copy(zeros_hbm, vbuf)
    done = 0
    while done < ROWS_PER_TILE:
        n = min(CHUNK, ROWS_PER_TILE - done)
        pltpu.sync_copy(vbuf.at[pl.ds(0, n)], acc.at[pl.ds(base + done, n)])
        done += n


def _make_agg(H):
    mesh = plsc.VectorSubcoreMesh(core_axis_name="c", subcore_axis_name="s")

    @functools.partial(
        pl.kernel,
        out_type=jax.ShapeDtypeStruct((NC, NPAD, H), jnp.float32),
        mesh=mesh,
        scratch_types=[
            pltpu.VMEM((NCMAX, CHUNK), jnp.int32),
            pltpu.VMEM((NCMAX, CHUNK), jnp.int32),
            [pltpu.VMEM((CHUNK, H), jnp.float32) for _ in range(NBUF)],
            pltpu.VMEM_SHARED((NPAD, H), jnp.float32),
            [pltpu.SemaphoreType.DMA for _ in range(NBUF)],
        ],
        compiler_params=pltpu.CompilerParams(use_tc_tiling_on_sc=False),
    )
    def agg(table, src_i, dst_i, zeros_hbm, out, sidx, didx, bufs, acc, sems):
        c = lax.axis_index("c")
        s = lax.axis_index("s")
        base = s * ROWS_PER_TILE
        _zero_spmem_slice(zeros_hbm, acc, bufs[0], base)
        n = jnp.where(c == 0, NC0, NC1)
        start = jnp.where(c == 0, s * NC0, NS * NC0 + s * NC1)
        pltpu.sync_copy(src_i.at[pl.ds(start, NCMAX)], sidx)
        pltpu.sync_copy(dst_i.at[pl.ds(start, NCMAX)], didx)
        plsc.subcore_barrier()

        for k in range(NBUF):
            pltpu.async_copy(table.at[sidx.at[k]], bufs[k], sems[k])

        def body(g, carry):
            j = g * NBUF
            for k in range(NBUF):
                pltpu.make_async_copy(table.at[sidx.at[j + k]], bufs[k],
                                      sems[k]).wait()
                pltpu.sync_copy(bufs[k], acc.at[didx.at[j + k]], add=True)

                @pl.when(j + k + NBUF < n)
                def _():
                    pltpu.async_copy(table.at[sidx.at[j + k + NBUF]], bufs[k],
                                     sems[k])

            return carry

        lax.fori_loop(0, n // NBUF, body, 0)
        plsc.subcore_barrier()
        _copy_spmem_slice_to_hbm(acc, out.at[c], bufs[0], base)

    return agg


DEGW = 8


def _make_deg():
    mesh = plsc.VectorSubcoreMesh(core_axis_name="c", subcore_axis_name="s")

    @functools.partial(
        pl.kernel,
        out_type=jax.ShapeDtypeStruct((NC, NPAD, DEGW), jnp.float32),
        mesh=mesh,
        scratch_types=[
            pltpu.VMEM((NCHUNK, CHUNK), jnp.int32),
            pltpu.VMEM((CHUNK, DEGW), jnp.float32),
            pltpu.VMEM((CHUNK, DEGW), jnp.float32),
            pltpu.VMEM_SHARED((NPAD, DEGW), jnp.float32),
        ],
        compiler_params=pltpu.CompilerParams(use_tc_tiling_on_sc=False),
    )
    def deg(dst_i, ones_hbm, zeros_hbm, out, didx, ones_v, vbuf, acc):
        c = lax.axis_index("c")
        s = lax.axis_index("s")
        wid = c * NS + s
        base = s * ROWS_PER_TILE
        _zero_spmem_slice(zeros_hbm, acc, vbuf, base)
        pltpu.sync_copy(ones_hbm, ones_v)
        pltpu.sync_copy(dst_i.at[wid], didx)
        plsc.subcore_barrier()

        def body(j, carry):
            pltpu.sync_copy(ones_v, acc.at[didx.at[j]], add=True)
            return carry

        lax.fori_loop(0, NCHUNK, body, 0)
        plsc.subcore_barrier()
        _copy_spmem_slice_to_hbm(acc, out.at[c], vbuf, base)

    return deg


_ROW2 = lambda shape: lax.broadcasted_iota(jnp.int32, shape, 0)


def _tc1_body(x_ref, w1_ref, degp_ref, hn1_ref, dinv_ref):
    deg = degp_ref[0, :, 0:1] + degp_ref[1, :, 0:1] + 1.0
    valid = _ROW2((NPAD, 1)) < N_NODES
    dinv = jnp.where(valid, lax.rsqrt(jnp.maximum(deg, 1e-12)), 0.0)
    h = jnp.dot(x_ref[...], w1_ref[...], preferred_element_type=jnp.float32)
    hn1_ref[...] = h * dinv
    dinv_ref[...] = dinv


def _bn_relu(conv, gamma, beta):
    valid = _ROW2(conv.shape) < N_NODES
    convm = jnp.where(valid, conv, 0.0)
    mean = jnp.sum(convm, axis=0, keepdims=True) / N_NODES
    dev = jnp.where(valid, conv - mean, 0.0)
    var = jnp.sum(dev * dev, axis=0, keepdims=True) / N_NODES
    y = (conv - mean) * lax.rsqrt(var + 1e-5) * gamma + beta
    return jnp.where(valid, jnp.maximum(y, 0.0), 0.0)


def _tc2_body(agg_ref, hn1_ref, dinv_ref, b1_ref, g1_ref, be1_ref, w2_ref,
              hn2_ref):
    dinv = dinv_ref[...]
    conv = (agg_ref[0] + agg_ref[1] + hn1_ref[...]) * dinv + b1_ref[...]
    y = _bn_relu(conv, g1_ref[...], be1_ref[...])
    h2 = jnp.dot(y, w2_ref[...], preferred_element_type=jnp.float32)
    hn2_ref[...] = h2 * dinv


def _tc3_body(agg_ref, hn2_ref, dinv_ref, b2_ref, g2_ref, be2_ref, wc_ref,
              bc_ref, out_ref):
    dinv = dinv_ref[...]
    conv = (agg_ref[0] + agg_ref[1] + hn2_ref[...]) * dinv + b2_ref[...]
    y = _bn_relu(conv, g2_ref[...], be2_ref[...])
    logits = jnp.dot(y, wc_ref[...], preferred_element_type=jnp.float32)
    logits = logits + bc_ref[...]
    m = jnp.max(logits, axis=1, keepdims=True)
    lse = jnp.log(jnp.sum(jnp.exp(logits - m), axis=1, keepdims=True)) + m
    out_ref[...] = logits - lse


_deg_call = _make_deg()
_agg64_call = _make_agg(H1)
_agg32_call = _make_agg(H2)

_tc1_call = pl.pallas_call(
    _tc1_body,
    out_shape=(
        jax.ShapeDtypeStruct((NPAD, H1), jnp.float32),
        jax.ShapeDtypeStruct((NPAD, 1), jnp.float32),
    ),
)

_tc2_call = pl.pallas_call(
    _tc2_body,
    out_shape=jax.ShapeDtypeStruct((NPAD, H2), jnp.float32),
)

_tc3_call = pl.pallas_call(
    _tc3_body,
    out_shape=jax.ShapeDtypeStruct((NPAD, C_OUT), jnp.float32),
)


def kernel(x, edge_index, W1, b1, gamma1, beta1, W2, b2, gamma2, beta2, Wc,
           bc):
    pad = E_PAD + SLACK * CHUNK - N_EDGES
    fill = jnp.full((pad,), N_NODES, dtype=jnp.int32)
    src = jnp.concatenate([edge_index[0], fill]).reshape(-1, CHUNK)
    dst = jnp.concatenate([edge_index[1], fill]).reshape(-1, CHUNK)
    dst3 = dst[:TOTAL_CHUNKS].reshape(NW, NCHUNK, CHUNK)

    ones1 = jnp.ones((CHUNK, DEGW), jnp.float32)
    zeros1 = jnp.zeros((CHUNK, DEGW), jnp.float32)
    zeros64 = jnp.zeros((CHUNK, H1), jnp.float32)
    zeros32 = jnp.zeros((CHUNK, H2), jnp.float32)

    degp = _deg_call(dst3, ones1, zeros1)

    xpad = jnp.pad(x, ((0, NPAD - N_NODES), (0, 0)))
    hn1, dinv = _tc1_call(xpad, W1, degp)

    agg1 = _agg64_call(hn1, src, dst, zeros64)
    hn2 = _tc2_call(agg1, hn1, dinv, b1.reshape(1, H1), gamma1.reshape(1, H1),
                    beta1.reshape(1, H1), W2)

    agg2 = _agg32_call(hn2, src, dst, zeros32)
    out = _tc3_call(agg2, hn2, dinv, b2.reshape(1, H2), gamma2.reshape(1, H2),
                    beta2.reshape(1, H2), Wc, bc.reshape(1, C_OUT))
    return out[:N_NODES]


# speedup vs baseline: 23.1541x; 1.0052x over previous
import functools

import jax
import jax.numpy as jnp
from jax import lax
from jax.experimental import pallas as pl
from jax.experimental.pallas import tpu as pltpu
from jax.experimental.pallas import tpu_sc as plsc

N_NODES = 10000
N_EDGES = 320000
D_IN = 128
H1 = 64
H2 = 32
C_OUT = 2

NC = 2
NS = 16
NW = NC * NS
CHUNK = 128
NCHUNK = 80
E_PAD = NW * NCHUNK * CHUNK
TOTAL_CHUNKS = E_PAD // CHUNK
NBUF = 4
AGG64_NC0 = 116
AGG32_NC0 = 112
SLACK = 160
NPAD = 10112
ROWS_PER_TILE = NPAD // NS


def _copy_spmem_slice_to_hbm(acc, out_c, vbuf, base):
    done = 0
    while done < ROWS_PER_TILE:
        n = min(CHUNK, ROWS_PER_TILE - done)
        off = base + done
        pltpu.sync_copy(acc.at[pl.ds(off, n)], vbuf.at[pl.ds(0, n)])
        pltpu.sync_copy(vbuf.at[pl.ds(0, n)], out_c.at[pl.ds(off, n)])
        done += n


def _zero_spmem_slice(zeros_hbm, acc, vbuf, base):
    pltpu.sync_copy(zeros_hbm, vbuf)
    done = 0
    while done < ROWS_PER_TILE:
        n = min(CHUNK, ROWS_PER_TILE - done)
        pltpu.sync_copy(vbuf.at[pl.ds(0, n)], acc.at[pl.ds(base + done, n)])
        done += n


def _make_agg(H, nc0):
    nc1 = 2 * NCHUNK - nc0
    ncmax = max(nc0, nc1)
    mesh = plsc.VectorSubcoreMesh(core_axis_name="c", subcore_axis_name="s")

    @functools.partial(
        pl.kernel,
        out_type=jax.ShapeDtypeStruct((NC, NPAD, H), jnp.float32),
        mesh=mesh,
        scratch_types=[
            pltpu.VMEM((ncmax, CHUNK), jnp.int32),
            pltpu.VMEM((ncmax, CHUNK), jnp.int32),
            [pltpu.VMEM((CHUNK, H), jnp.float32) for _ in range(NBUF)],
            pltpu.VMEM_SHARED((NPAD, H), jnp.float32),
            [pltpu.SemaphoreType.DMA for _ in range(NBUF)],
        ],
        compiler_params=pltpu.CompilerParams(use_tc_tiling_on_sc=False),
    )
    def agg(table, src_i, dst_i, zeros_hbm, out, sidx, didx, bufs, acc, sems):
        c = lax.axis_index("c")
        s = lax.axis_index("s")
        base = s * ROWS_PER_TILE
        _zero_spmem_slice(zeros_hbm, acc, bufs[0], base)
        n = jnp.where(c == 0, nc0, nc1)
        start = jnp.where(c == 0, s * nc0, NS * nc0 + s * nc1)
        pltpu.sync_copy(src_i.at[pl.ds(start, ncmax)], sidx)
        pltpu.sync_copy(dst_i.at[pl.ds(start, ncmax)], didx)
        plsc.subcore_barrier()

        for k in range(NBUF):
            pltpu.async_copy(table.at[sidx.at[k]], bufs[k], sems[k])

        def body(g, carry):
            j = g * NBUF
            for k in range(NBUF):
                pltpu.make_async_copy(table.at[sidx.at[j + k]], bufs[k],
                                      sems[k]).wait()
                pltpu.sync_copy(bufs[k], acc.at[didx.at[j + k]], add=True)

                @pl.when(j + k + NBUF < n)
                def _():
                    pltpu.async_copy(table.at[sidx.at[j + k + NBUF]], bufs[k],
                                     sems[k])

            return carry

        lax.fori_loop(0, n // NBUF, body, 0)
        plsc.subcore_barrier()
        _copy_spmem_slice_to_hbm(acc, out.at[c], bufs[0], base)

    return agg


DEGW = 8


def _make_deg():
    mesh = plsc.VectorSubcoreMesh(core_axis_name="c", subcore_axis_name="s")

    @functools.partial(
        pl.kernel,
        out_type=jax.ShapeDtypeStruct((NC, NPAD, DEGW), jnp.float32),
        mesh=mesh,
        scratch_types=[
            pltpu.VMEM((NCHUNK, CHUNK), jnp.int32),
            pltpu.VMEM((CHUNK, DEGW), jnp.float32),
            pltpu.VMEM((CHUNK, DEGW), jnp.float32),
            pltpu.VMEM_SHARED((NPAD, DEGW), jnp.float32),
        ],
        compiler_params=pltpu.CompilerParams(use_tc_tiling_on_sc=False),
    )
    def deg(dst_i, ones_hbm, zeros_hbm, out, didx, ones_v, vbuf, acc):
        c = lax.axis_index("c")
        s = lax.axis_index("s")
        wid = c * NS + s
        base = s * ROWS_PER_TILE
        _zero_spmem_slice(zeros_hbm, acc, vbuf, base)
        pltpu.sync_copy(ones_hbm, ones_v)
        pltpu.sync_copy(dst_i.at[wid], didx)
        plsc.subcore_barrier()

        def body(j, carry):
            pltpu.sync_copy(ones_v, acc.at[didx.at[j]], add=True)
            return carry

        lax.fori_loop(0, NCHUNK, body, 0)
        plsc.subcore_barrier()
        _copy_spmem_slice_to_hbm(acc, out.at[c], vbuf, base)

    return deg


_ROW2 = lambda shape: lax.broadcasted_iota(jnp.int32, shape, 0)


def _tc1_body(x_ref, w1_ref, degp_ref, hn1_ref, dinv_ref):
    deg = degp_ref[0, :, 0:1] + degp_ref[1, :, 0:1] + 1.0
    valid = _ROW2((NPAD, 1)) < N_NODES
    dinv = jnp.where(valid, lax.rsqrt(jnp.maximum(deg, 1e-12)), 0.0)
    h = jnp.dot(x_ref[...], w1_ref[...], preferred_element_type=jnp.float32)
    hn1_ref[...] = h * dinv
    dinv_ref[...] = dinv


def _bn_relu(conv, gamma, beta):
    valid = _ROW2(conv.shape) < N_NODES
    convm = jnp.where(valid, conv, 0.0)
    mean = jnp.sum(convm, axis=0, keepdims=True) / N_NODES
    dev = jnp.where(valid, conv - mean, 0.0)
    var = jnp.sum(dev * dev, axis=0, keepdims=True) / N_NODES
    y = (conv - mean) * lax.rsqrt(var + 1e-5) * gamma + beta
    return jnp.where(valid, jnp.maximum(y, 0.0), 0.0)


def _tc2_body(agg_ref, hn1_ref, dinv_ref, b1_ref, g1_ref, be1_ref, w2_ref,
              hn2_ref):
    dinv = dinv_ref[...]
    conv = (agg_ref[0] + agg_ref[1] + hn1_ref[...]) * dinv + b1_ref[...]
    y = _bn_relu(conv, g1_ref[...], be1_ref[...])
    h2 = jnp.dot(y, w2_ref[...], preferred_element_type=jnp.float32)
    hn2_ref[...] = h2 * dinv


def _tc3_body(agg_ref, hn2_ref, dinv_ref, b2_ref, g2_ref, be2_ref, wc_ref,
              bc_ref, out_ref):
    dinv = dinv_ref[...]
    conv = (agg_ref[0] + agg_ref[1] + hn2_ref[...]) * dinv + b2_ref[...]
    y = _bn_relu(conv, g2_ref[...], be2_ref[...])
    logits = jnp.dot(y, wc_ref[...], preferred_element_type=jnp.float32)
    logits = logits + bc_ref[...]
    m = jnp.max(logits, axis=1, keepdims=True)
    lse = jnp.log(jnp.sum(jnp.exp(logits - m), axis=1, keepdims=True)) + m
    out_ref[...] = logits - lse


_deg_call = _make_deg()
_agg64_call = _make_agg(H1, AGG64_NC0)
_agg32_call = _make_agg(H2, AGG32_NC0)

_tc1_call = pl.pallas_call(
    _tc1_body,
    out_shape=(
        jax.ShapeDtypeStruct((NPAD, H1), jnp.float32),
        jax.ShapeDtypeStruct((NPAD, 1), jnp.float32),
    ),
)

_tc2_call = pl.pallas_call(
    _tc2_body,
    out_shape=jax.ShapeDtypeStruct((NPAD, H2), jnp.float32),
)

_tc3_call = pl.pallas_call(
    _tc3_body,
    out_shape=jax.ShapeDtypeStruct((NPAD, C_OUT), jnp.float32),
)


def kernel(x, edge_index, W1, b1, gamma1, beta1, W2, b2, gamma2, beta2, Wc,
           bc):
    pad = E_PAD + SLACK * CHUNK - N_EDGES
    fill = jnp.full((pad,), N_NODES, dtype=jnp.int32)
    src = jnp.concatenate([edge_index[0], fill]).reshape(-1, CHUNK)
    dst = jnp.concatenate([edge_index[1], fill]).reshape(-1, CHUNK)
    dst3 = dst[:TOTAL_CHUNKS].reshape(NW, NCHUNK, CHUNK)

    ones1 = jnp.ones((CHUNK, DEGW), jnp.float32)
    zeros1 = jnp.zeros((CHUNK, DEGW), jnp.float32)
    zeros64 = jnp.zeros((CHUNK, H1), jnp.float32)
    zeros32 = jnp.zeros((CHUNK, H2), jnp.float32)

    degp = _deg_call(dst3, ones1, zeros1)

    xpad = jnp.pad(x, ((0, NPAD - N_NODES), (0, 0)))
    hn1, dinv = _tc1_call(xpad, W1, degp)

    agg1 = _agg64_call(hn1, src, dst, zeros64)
    hn2 = _tc2_call(agg1, hn1, dinv, b1.reshape(1, H1), gamma1.reshape(1, H1),
                    beta1.reshape(1, H1), W2)

    agg2 = _agg32_call(hn2, src, dst, zeros32)
    out = _tc3_call(agg2, hn2, dinv, b2.reshape(1, H2), gamma2.reshape(1, H2),
                    beta2.reshape(1, H2), Wc, bc.reshape(1, C_OUT))
    return out[:N_NODES]
